# baseline (speedup 1.0000x reference)
import functools
import jax
import jax.numpy as jnp
from jax import lax
from jax.experimental import pallas as pl
from jax.experimental.pallas import tpu as pltpu
from jax.experimental.pallas import tpu_sc as plsc

N_NODES = 10000
N_EDGES = 320000
D_IN = 128
D_HID = 128
D_OUT = 64

NP = 10240
NW = 32
CHUNK = 128
CPT = 80
EPT = CPT * CHUNK
EP = NW * EPT
HCPT = CPT // 2
RPT = NP // 16
ROWB = 400
GRID = N_NODES // ROWB


def _mesh():
    return plsc.VectorSubcoreMesh(core_axis_name="c", subcore_axis_name="s")


def _zero_vmem(buf, nrows, ncols):
    zeros = jnp.zeros((16,), jnp.float32)

    def zrow(i, _):
        def zcol(j, _):
            buf[i, pl.ds(j * 16, 16)] = zeros
            return 0
        return lax.fori_loop(0, ncols // 16, zcol, 0)

    lax.fori_loop(0, nrows, zrow, 0)


def _hist_body(dst_hbm, out_hbm, acc, dst_v, buf):
    c = lax.axis_index("c")
    s = lax.axis_index("s")
    wid = c * 16 + s
    base = s * RPT
    _zero_vmem(buf, CHUNK, 16)

    def zacc(k, _):
        pltpu.sync_copy(buf, acc.at[pl.ds(base + k * CHUNK, CHUNK)])
        return 0
    lax.fori_loop(0, RPT // CHUNK, zacc, 0)

    ones = jnp.ones((16,), jnp.float32)

    def orow(i, _):
        buf[i, pl.ds(0, 16)] = ones
        return 0
    lax.fori_loop(0, CHUNK, orow, 0)

    pltpu.sync_copy(dst_hbm.at[wid], dst_v)
    plsc.subcore_barrier()

    def step(j, _):
        pltpu.sync_copy(buf, acc.at[dst_v.at[j]], add=True)
        return 0
    lax.fori_loop(0, CPT, step, 0)
    plsc.subcore_barrier()

    def wout(k, _):
        pltpu.sync_copy(acc.at[pl.ds(base + k * CHUNK, CHUNK)],
                        out_hbm.at[c, pl.ds(base + k * CHUNK, CHUNK)])
        return 0
    lax.fori_loop(0, RPT // CHUNK, wout, 0)


@functools.partial(
    pl.kernel,
    out_type=jax.ShapeDtypeStruct((2, NP, 16), jnp.float32),
    mesh=_mesh(),
    scratch_types=[
        pltpu.VMEM_SHARED((NP, 16), jnp.float32),
        pltpu.VMEM((CPT, CHUNK), jnp.int32),
        pltpu.VMEM((CHUNK, 16), jnp.float32),
    ],
)
def _hist(dst_hbm, out_hbm, acc, dst_v, buf):
    _hist_body(dst_hbm, out_hbm, acc, dst_v, buf)


def _agg_body(g_hbm, src_hbm, dst_hbm, out_hbm, acc,
              src_v, dst_v, rows0, rows1, gs0, gs1, ss0, ss1):
    c = lax.axis_index("c")
    s = lax.axis_index("s")
    wid = c * 16 + s
    base = s * RPT
    _zero_vmem(rows0, CHUNK, D_HID)

    def zacc(k, _):
        pltpu.sync_copy(rows0, acc.at[pl.ds(base + k * CHUNK, CHUNK)])
        return 0
    lax.fori_loop(0, RPT // CHUNK, zacc, 0)

    plsc.subcore_barrier()

    def half(h, _):
        pltpu.sync_copy(src_hbm.at[wid, pl.ds(h * HCPT, HCPT)], src_v)
        pltpu.sync_copy(dst_hbm.at[wid, pl.ds(h * HCPT, HCPT)], dst_v)
        pltpu.async_copy(g_hbm.at[src_v.at[0]], rows0, gs0)
        pltpu.async_copy(g_hbm.at[src_v.at[1]], rows1, gs1)

        def step(i, _):
            j0 = 2 * i
            pltpu.make_async_copy(g_hbm.at[src_v.at[j0]], rows0, gs0).wait()
            pltpu.sync_copy(rows0, acc.at[dst_v.at[j0]], add=True)

            @pl.when(i < HCPT // 2 - 1)
            def _():
                pltpu.async_copy(g_hbm.at[src_v.at[j0 + 2]], rows0, gs0)

            pltpu.make_async_copy(g_hbm.at[src_v.at[j0 + 1]], rows1, gs1).wait()
            pltpu.sync_copy(rows1, acc.at[dst_v.at[j0 + 1]], add=True)

            @pl.when(i < HCPT // 2 - 1)
            def _():
                pltpu.async_copy(g_hbm.at[src_v.at[j0 + 3]], rows1, gs1)
            return 0
        lax.fori_loop(0, HCPT // 2, step, 0)
        return 0
    lax.fori_loop(0, 2, half, 0)
    plsc.subcore_barrier()

    def wout(k, _):
        pltpu.sync_copy(acc.at[pl.ds(base + k * CHUNK, CHUNK)],
                        out_hbm.at[c, pl.ds(base + k * CHUNK, CHUNK)])
        return 0
    lax.fori_loop(0, RPT // CHUNK, wout, 0)


@functools.partial(
    pl.kernel,
    out_type=jax.ShapeDtypeStruct((2, NP, D_HID), jnp.float32),
    mesh=_mesh(),
    scratch_types=[
        pltpu.VMEM_SHARED((NP, D_HID), jnp.float32),
        pltpu.VMEM((HCPT, CHUNK), jnp.int32),
        pltpu.VMEM((HCPT, CHUNK), jnp.int32),
        pltpu.VMEM((CHUNK, D_HID), jnp.float32),
        pltpu.VMEM((CHUNK, D_HID), jnp.float32),
        pltpu.SemaphoreType.DMA,
        pltpu.SemaphoreType.DMA,
        pltpu.SemaphoreType.DMA,
        pltpu.SemaphoreType.DMA,
    ],
)
def _agg(g_hbm, src_hbm, dst_hbm, out_hbm, acc,
         src_v, dst_v, rows0, rows1, gs0, gs1, ss0, ss1):
    _agg_body(g_hbm, src_hbm, dst_hbm, out_hbm, acc,
              src_v, dst_v, rows0, rows1, gs0, gs1, ss0, ss1)


def _dis_of(h_ref):
    deg = h_ref[0, :, 0:1] + h_ref[1, :, 0:1] + 1.0
    return lax.rsqrt(deg)


def _k1_body(x_ref, w_ref, h_ref, g_ref):
    dis = _dis_of(h_ref)
    p = jnp.dot(x_ref[...], w_ref[...], preferred_element_type=jnp.float32)
    g_ref[...] = dis * p


def _k23_body(s_ref, g_ref, h_ref, w_ref, b_ref, gout_ref, apply_w):
    dis = _dis_of(h_ref)
    agg = s_ref[0] + s_ref[1]
    h = jnp.maximum(dis * agg + dis * g_ref[...] + b_ref[...], 0.0)
    if apply_w:
        pnew = jnp.dot(h, w_ref[...], preferred_element_type=jnp.float32)
    else:
        pnew = h
    gout_ref[...] = dis * pnew


def _k4_body(s_ref, g_ref, h_ref, wmu_ref, bmu_ref, wlv_ref, blv_ref,
             mu_ref, lv_ref):
    dis = _dis_of(h_ref)
    agg = s_ref[0] + s_ref[1]
    a3 = dis * agg + dis * g_ref[...]
    mu_ref[...] = jnp.dot(a3, wmu_ref[...], preferred_element_type=jnp.float32) + bmu_ref[...]
    lv_ref[...] = jnp.dot(a3, wlv_ref[...], preferred_element_type=jnp.float32) + blv_ref[...]


def _row_spec(width):
    return pl.BlockSpec((ROWB, width), lambda i: (i, 0))


def _part_spec(width):
    return pl.BlockSpec((2, ROWB, width), lambda i: (0, i, 0))


def _full_spec(r, cmax):
    return pl.BlockSpec((r, cmax), lambda i: (0, 0))


@jax.jit
def kernel(x, edge_index, W1, b1, W2, b2, W_mu, b_mu, W_logvar, b_logvar):
    src = edge_index[0]
    dst = edge_index[1]
    pad = (N_NODES
           + jnp.arange(EP - N_EDGES, dtype=jnp.int32) % (NP - N_NODES))
    src3 = jnp.concatenate([src, pad]).reshape(NW, CPT, CHUNK)
    dst3 = jnp.concatenate([dst, pad]).reshape(NW, CPT, CHUNK)
    b1r = b1.reshape(1, -1)
    b2r = b2.reshape(1, -1)
    bmur = b_mu.reshape(1, -1)
    blvr = b_logvar.reshape(1, -1)

    hist = _hist(dst3)

    g1 = pl.pallas_call(
        _k1_body,
        grid=(GRID,),
        in_specs=[_row_spec(D_IN), _full_spec(D_IN, D_HID), _part_spec(16)],
        out_specs=_row_spec(D_HID),
        out_shape=jax.ShapeDtypeStruct((NP, D_HID), jnp.float32),
    )(x, W1, hist)

    s1 = _agg(g1, src3, dst3)

    g2 = pl.pallas_call(
        functools.partial(_k23_body, apply_w=True),
        grid=(GRID,),
        in_specs=[_part_spec(D_HID), _row_spec(D_HID), _part_spec(16),
                  _full_spec(D_HID, D_HID), _full_spec(1, D_HID)],
        out_specs=_row_spec(D_HID),
        out_shape=jax.ShapeDtypeStruct((NP, D_HID), jnp.float32),
    )(s1, g1, hist, W2, b1r)

    s2 = _agg(g2, src3, dst3)

    g3 = pl.pallas_call(
        functools.partial(_k23_body, apply_w=False),
        grid=(GRID,),
        in_specs=[_part_spec(D_HID), _row_spec(D_HID), _part_spec(16),
                  _full_spec(D_HID, D_HID), _full_spec(1, D_HID)],
        out_specs=_row_spec(D_HID),
        out_shape=jax.ShapeDtypeStruct((NP, D_HID), jnp.float32),
    )(s2, g2, hist, W2, b2r)

    s3 = _agg(g3, src3, dst3)

    mu, lv = pl.pallas_call(
        _k4_body,
        grid=(GRID,),
        in_specs=[_part_spec(D_HID), _row_spec(D_HID), _part_spec(16),
                  _full_spec(D_HID, D_OUT), _full_spec(1, D_OUT),
                  _full_spec(D_HID, D_OUT), _full_spec(1, D_OUT)],
        out_specs=[_row_spec(D_OUT), _row_spec(D_OUT)],
        out_shape=[jax.ShapeDtypeStruct((N_NODES, D_OUT), jnp.float32)] * 2,
    )(s3, g3, hist, W_mu, bmur, W_logvar, blvr)

    return (mu, lv)


# speedup vs baseline: 1.0724x; 1.0724x over previous
import functools
import jax
import jax.numpy as jnp
from jax import lax
from jax.experimental import pallas as pl
from jax.experimental.pallas import tpu as pltpu
from jax.experimental.pallas import tpu_sc as plsc

N_NODES = 10000
N_EDGES = 320000
D_IN = 128
D_HID = 128
D_OUT = 64

NP = 10240
NW = 32
CHUNK = 128
CPT = 80
EPT = CPT * CHUNK
EP = NW * EPT
HCPT = CPT // 2
RPT = NP // 16
ROWB = 1000
GRID = N_NODES // ROWB


def _mesh():
    return plsc.VectorSubcoreMesh(core_axis_name="c", subcore_axis_name="s")


def _zero_vmem(buf, nrows, ncols):
    zeros = jnp.zeros((16,), jnp.float32)

    def zrow(i, _):
        def zcol(j, _):
            buf[i, pl.ds(j * 16, 16)] = zeros
            return 0
        return lax.fori_loop(0, ncols // 16, zcol, 0)

    lax.fori_loop(0, nrows, zrow, 0)


def _hist_body(dst_hbm, out_hbm, acc, dst_v, buf):
    c = lax.axis_index("c")
    s = lax.axis_index("s")
    wid = c * 16 + s
    base = s * RPT
    _zero_vmem(buf, CHUNK, 16)

    def zacc(k, _):
        pltpu.sync_copy(buf, acc.at[pl.ds(base + k * CHUNK, CHUNK)])
        return 0
    lax.fori_loop(0, RPT // CHUNK, zacc, 0)

    ones = jnp.ones((16,), jnp.float32)

    def orow(i, _):
        buf[i, pl.ds(0, 16)] = ones
        return 0
    lax.fori_loop(0, CHUNK, orow, 0)

    pltpu.sync_copy(dst_hbm.at[wid], dst_v)
    plsc.subcore_barrier()

    def step(j, _):
        pltpu.sync_copy(buf, acc.at[dst_v.at[j]], add=True)
        return 0
    lax.fori_loop(0, CPT, step, 0)
    plsc.subcore_barrier()

    def wout(k, _):
        pltpu.sync_copy(acc.at[pl.ds(base + k * CHUNK, CHUNK)],
                        out_hbm.at[c, pl.ds(base + k * CHUNK, CHUNK)])
        return 0
    lax.fori_loop(0, RPT // CHUNK, wout, 0)


@functools.partial(
    pl.kernel,
    out_type=jax.ShapeDtypeStruct((2, NP, 16), jnp.float32),
    mesh=_mesh(),
    scratch_types=[
        pltpu.VMEM_SHARED((NP, 16), jnp.float32),
        pltpu.VMEM((CPT, CHUNK), jnp.int32),
        pltpu.VMEM((CHUNK, 16), jnp.float32),
    ],
)
def _hist(dst_hbm, out_hbm, acc, dst_v, buf):
    _hist_body(dst_hbm, out_hbm, acc, dst_v, buf)


def _agg_body(g_hbm, src_hbm, dst_hbm, out_hbm, acc,
              src_v, dst_v, rows0, rows1, gs0, gs1, ss0, ss1):
    c = lax.axis_index("c")
    s = lax.axis_index("s")
    wid = c * 16 + s
    base = s * RPT
    _zero_vmem(rows0, CHUNK, D_HID)

    def zacc(k, _):
        pltpu.sync_copy(rows0, acc.at[pl.ds(base + k * CHUNK, CHUNK)])
        return 0
    lax.fori_loop(0, RPT // CHUNK, zacc, 0)

    plsc.subcore_barrier()

    def half(h, _):
        pltpu.sync_copy(src_hbm.at[wid, pl.ds(h * HCPT, HCPT)], src_v)
        pltpu.sync_copy(dst_hbm.at[wid, pl.ds(h * HCPT, HCPT)], dst_v)
        pltpu.async_copy(g_hbm.at[src_v.at[0]], rows0, gs0)
        pltpu.async_copy(g_hbm.at[src_v.at[1]], rows1, gs1)

        def step(i, _):
            j0 = 2 * i
            pltpu.make_async_copy(g_hbm.at[src_v.at[j0]], rows0, gs0).wait()
            pltpu.sync_copy(rows0, acc.at[dst_v.at[j0]], add=True)

            @pl.when(i < HCPT // 2 - 1)
            def _():
                pltpu.async_copy(g_hbm.at[src_v.at[j0 + 2]], rows0, gs0)

            pltpu.make_async_copy(g_hbm.at[src_v.at[j0 + 1]], rows1, gs1).wait()
            pltpu.sync_copy(rows1, acc.at[dst_v.at[j0 + 1]], add=True)

            @pl.when(i < HCPT // 2 - 1)
            def _():
                pltpu.async_copy(g_hbm.at[src_v.at[j0 + 3]], rows1, gs1)
            return 0
        lax.fori_loop(0, HCPT // 2, step, 0)
        return 0
    lax.fori_loop(0, 2, half, 0)
    plsc.subcore_barrier()

    def wout(k, _):
        pltpu.sync_copy(acc.at[pl.ds(base + k * CHUNK, CHUNK)],
                        out_hbm.at[c, pl.ds(base + k * CHUNK, CHUNK)])
        return 0
    lax.fori_loop(0, RPT // CHUNK, wout, 0)


@functools.partial(
    pl.kernel,
    out_type=jax.ShapeDtypeStruct((2, NP, D_HID), jnp.float32),
    mesh=_mesh(),
    scratch_types=[
        pltpu.VMEM_SHARED((NP, D_HID), jnp.float32),
        pltpu.VMEM((HCPT, CHUNK), jnp.int32),
        pltpu.VMEM((HCPT, CHUNK), jnp.int32),
        pltpu.VMEM((CHUNK, D_HID), jnp.float32),
        pltpu.VMEM((CHUNK, D_HID), jnp.float32),
        pltpu.SemaphoreType.DMA,
        pltpu.SemaphoreType.DMA,
        pltpu.SemaphoreType.DMA,
        pltpu.SemaphoreType.DMA,
    ],
)
def _agg(g_hbm, src_hbm, dst_hbm, out_hbm, acc,
         src_v, dst_v, rows0, rows1, gs0, gs1, ss0, ss1):
    _agg_body(g_hbm, src_hbm, dst_hbm, out_hbm, acc,
              src_v, dst_v, rows0, rows1, gs0, gs1, ss0, ss1)


def _dis_of(h_ref):
    deg = h_ref[0, :, 0:1] + h_ref[1, :, 0:1] + 1.0
    return lax.rsqrt(deg)


def _k1_body(x_ref, w_ref, h_ref, g_ref):
    dis = _dis_of(h_ref)
    p = jnp.dot(x_ref[...], w_ref[...], preferred_element_type=jnp.float32)
    g_ref[...] = dis * p


def _k23_body(s_ref, g_ref, h_ref, w_ref, b_ref, gout_ref, apply_w):
    dis = _dis_of(h_ref)
    agg = s_ref[0] + s_ref[1]
    h = jnp.maximum(dis * agg + dis * g_ref[...] + b_ref[...], 0.0)
    if apply_w:
        pnew = jnp.dot(h, w_ref[...], preferred_element_type=jnp.float32)
    else:
        pnew = h
    gout_ref[...] = dis * pnew


def _k4_body(s_ref, g_ref, h_ref, wmu_ref, bmu_ref, wlv_ref, blv_ref,
             mu_ref, lv_ref):
    dis = _dis_of(h_ref)
    agg = s_ref[0] + s_ref[1]
    a3 = dis * agg + dis * g_ref[...]
    mu_ref[...] = jnp.dot(a3, wmu_ref[...], preferred_element_type=jnp.float32) + bmu_ref[...]
    lv_ref[...] = jnp.dot(a3, wlv_ref[...], preferred_element_type=jnp.float32) + blv_ref[...]


def _row_spec(width):
    return pl.BlockSpec((ROWB, width), lambda i: (i, 0))


def _part_spec(width):
    return pl.BlockSpec((2, ROWB, width), lambda i: (0, i, 0))


def _full_spec(r, cmax):
    return pl.BlockSpec((r, cmax), lambda i: (0, 0))


@jax.jit
def kernel(x, edge_index, W1, b1, W2, b2, W_mu, b_mu, W_logvar, b_logvar):
    src = edge_index[0]
    dst = edge_index[1]
    pad = (N_NODES
           + jnp.arange(EP - N_EDGES, dtype=jnp.int32) % (NP - N_NODES))
    src3 = jnp.concatenate([src, pad]).reshape(NW, CPT, CHUNK)
    dst3 = jnp.concatenate([dst, pad]).reshape(NW, CPT, CHUNK)
    b1r = b1.reshape(1, -1)
    b2r = b2.reshape(1, -1)
    bmur = b_mu.reshape(1, -1)
    blvr = b_logvar.reshape(1, -1)

    hist = _hist(dst3)

    g1 = pl.pallas_call(
        _k1_body,
        grid=(GRID,),
        in_specs=[_row_spec(D_IN), _full_spec(D_IN, D_HID), _part_spec(16)],
        out_specs=_row_spec(D_HID),
        out_shape=jax.ShapeDtypeStruct((NP, D_HID), jnp.float32),
    )(x, W1, hist)

    s1 = _agg(g1, src3, dst3)

    g2 = pl.pallas_call(
        functools.partial(_k23_body, apply_w=True),
        grid=(GRID,),
        in_specs=[_part_spec(D_HID), _row_spec(D_HID), _part_spec(16),
                  _full_spec(D_HID, D_HID), _full_spec(1, D_HID)],
        out_specs=_row_spec(D_HID),
        out_shape=jax.ShapeDtypeStruct((NP, D_HID), jnp.float32),
    )(s1, g1, hist, W2, b1r)

    s2 = _agg(g2, src3, dst3)

    g3 = pl.pallas_call(
        functools.partial(_k23_body, apply_w=False),
        grid=(GRID,),
        in_specs=[_part_spec(D_HID), _row_spec(D_HID), _part_spec(16),
                  _full_spec(D_HID, D_HID), _full_spec(1, D_HID)],
        out_specs=_row_spec(D_HID),
        out_shape=jax.ShapeDtypeStruct((NP, D_HID), jnp.float32),
    )(s2, g2, hist, W2, b2r)

    s3 = _agg(g3, src3, dst3)

    mu, lv = pl.pallas_call(
        _k4_body,
        grid=(GRID,),
        in_specs=[_part_spec(D_HID), _row_spec(D_HID), _part_spec(16),
                  _full_spec(D_HID, D_OUT), _full_spec(1, D_OUT),
                  _full_spec(D_HID, D_OUT), _full_spec(1, D_OUT)],
        out_specs=[_row_spec(D_OUT), _row_spec(D_OUT)],
        out_shape=[jax.ShapeDtypeStruct((N_NODES, D_OUT), jnp.float32)] * 2,
    )(s3, g3, hist, W_mu, bmur, W_logvar, blvr)

    return (mu, lv)


# speedup vs baseline: 1.0923x; 1.0186x over previous
import functools
import jax
import jax.numpy as jnp
from jax import lax
from jax.experimental import pallas as pl
from jax.experimental.pallas import tpu as pltpu
from jax.experimental.pallas import tpu_sc as plsc

N_NODES = 10000
N_EDGES = 320000
D_IN = 128
D_HID = 128
D_OUT = 64

NP = 10240
NW = 32
CHUNK = 128
CPT = 80
EPT = CPT * CHUNK
EP = NW * EPT
HCPT = CPT // 2
RPT = NP // 16
ROWB = 2000
GRID = N_NODES // ROWB


def _mesh():
    return plsc.VectorSubcoreMesh(core_axis_name="c", subcore_axis_name="s")


def _zero_vmem(buf, nrows, ncols):
    zeros = jnp.zeros((16,), jnp.float32)

    def zrow(i, _):
        def zcol(j, _):
            buf[i, pl.ds(j * 16, 16)] = zeros
            return 0
        return lax.fori_loop(0, ncols // 16, zcol, 0)

    lax.fori_loop(0, nrows, zrow, 0)


def _hist_body(dst_hbm, out_hbm, acc, dst_v, buf):
    c = lax.axis_index("c")
    s = lax.axis_index("s")
    wid = c * 16 + s
    base = s * RPT
    _zero_vmem(buf, CHUNK, 16)

    def zacc(k, _):
        pltpu.sync_copy(buf, acc.at[pl.ds(base + k * CHUNK, CHUNK)])
        return 0
    lax.fori_loop(0, RPT // CHUNK, zacc, 0)

    ones = jnp.ones((16,), jnp.float32)

    def orow(i, _):
        buf[i, pl.ds(0, 16)] = ones
        return 0
    lax.fori_loop(0, CHUNK, orow, 0)

    pltpu.sync_copy(dst_hbm.at[wid], dst_v)
    plsc.subcore_barrier()

    def step(j, _):
        pltpu.sync_copy(buf, acc.at[dst_v.at[j]], add=True)
        return 0
    lax.fori_loop(0, CPT, step, 0)
    plsc.subcore_barrier()

    def wout(k, _):
        pltpu.sync_copy(acc.at[pl.ds(base + k * CHUNK, CHUNK)],
                        out_hbm.at[c, pl.ds(base + k * CHUNK, CHUNK)])
        return 0
    lax.fori_loop(0, RPT // CHUNK, wout, 0)


@functools.partial(
    pl.kernel,
    out_type=jax.ShapeDtypeStruct((2, NP, 16), jnp.float32),
    mesh=_mesh(),
    scratch_types=[
        pltpu.VMEM_SHARED((NP, 16), jnp.float32),
        pltpu.VMEM((CPT, CHUNK), jnp.int32),
        pltpu.VMEM((CHUNK, 16), jnp.float32),
    ],
)
def _hist(dst_hbm, out_hbm, acc, dst_v, buf):
    _hist_body(dst_hbm, out_hbm, acc, dst_v, buf)


def _agg_body(g_hbm, src_hbm, dst_hbm, out_hbm, acc,
              src_v, dst_v, rows0, rows1, gs0, gs1, ss0, ss1):
    c = lax.axis_index("c")
    s = lax.axis_index("s")
    wid = c * 16 + s
    base = s * RPT
    _zero_vmem(rows0, CHUNK, D_HID)

    def zacc(k, _):
        pltpu.sync_copy(rows0, acc.at[pl.ds(base + k * CHUNK, CHUNK)])
        return 0
    lax.fori_loop(0, RPT // CHUNK, zacc, 0)

    plsc.subcore_barrier()

    def half(h, _):
        pltpu.sync_copy(src_hbm.at[wid, pl.ds(h * HCPT, HCPT)], src_v)
        pltpu.sync_copy(dst_hbm.at[wid, pl.ds(h * HCPT, HCPT)], dst_v)
        pltpu.async_copy(g_hbm.at[src_v.at[0]], rows0, gs0)
        pltpu.async_copy(g_hbm.at[src_v.at[1]], rows1, gs1)

        def step(i, _):
            j0 = 2 * i
            pltpu.make_async_copy(g_hbm.at[src_v.at[j0]], rows0, gs0).wait()
            pltpu.sync_copy(rows0, acc.at[dst_v.at[j0]], add=True)

            @pl.when(i < HCPT // 2 - 1)
            def _():
                pltpu.async_copy(g_hbm.at[src_v.at[j0 + 2]], rows0, gs0)

            pltpu.make_async_copy(g_hbm.at[src_v.at[j0 + 1]], rows1, gs1).wait()
            pltpu.sync_copy(rows1, acc.at[dst_v.at[j0 + 1]], add=True)

            @pl.when(i < HCPT // 2 - 1)
            def _():
                pltpu.async_copy(g_hbm.at[src_v.at[j0 + 3]], rows1, gs1)
            return 0
        lax.fori_loop(0, HCPT // 2, step, 0)
        return 0
    lax.fori_loop(0, 2, half, 0)
    plsc.subcore_barrier()

    def wout(k, _):
        pltpu.sync_copy(acc.at[pl.ds(base + k * CHUNK, CHUNK)],
                        out_hbm.at[c, pl.ds(base + k * CHUNK, CHUNK)])
        return 0
    lax.fori_loop(0, RPT // CHUNK, wout, 0)


@functools.partial(
    pl.kernel,
    out_type=jax.ShapeDtypeStruct((2, NP, D_HID), jnp.float32),
    mesh=_mesh(),
    scratch_types=[
        pltpu.VMEM_SHARED((NP, D_HID), jnp.float32),
        pltpu.VMEM((HCPT, CHUNK), jnp.int32),
        pltpu.VMEM((HCPT, CHUNK), jnp.int32),
        pltpu.VMEM((CHUNK, D_HID), jnp.float32),
        pltpu.VMEM((CHUNK, D_HID), jnp.float32),
        pltpu.SemaphoreType.DMA,
        pltpu.SemaphoreType.DMA,
        pltpu.SemaphoreType.DMA,
        pltpu.SemaphoreType.DMA,
    ],
)
def _agg(g_hbm, src_hbm, dst_hbm, out_hbm, acc,
         src_v, dst_v, rows0, rows1, gs0, gs1, ss0, ss1):
    _agg_body(g_hbm, src_hbm, dst_hbm, out_hbm, acc,
              src_v, dst_v, rows0, rows1, gs0, gs1, ss0, ss1)


def _dis_of(h_ref):
    deg = h_ref[0, :, 0:1] + h_ref[1, :, 0:1] + 1.0
    return lax.rsqrt(deg)


def _k1_body(x_ref, w_ref, h_ref, g_ref):
    dis = _dis_of(h_ref)
    p = jnp.dot(x_ref[...], w_ref[...], preferred_element_type=jnp.float32)
    g_ref[...] = dis * p


def _k23_body(s_ref, g_ref, h_ref, w_ref, b_ref, gout_ref, apply_w):
    dis = _dis_of(h_ref)
    agg = s_ref[0] + s_ref[1]
    h = jnp.maximum(dis * agg + dis * g_ref[...] + b_ref[...], 0.0)
    if apply_w:
        pnew = jnp.dot(h, w_ref[...], preferred_element_type=jnp.float32)
    else:
        pnew = h
    gout_ref[...] = dis * pnew


def _k4_body(s_ref, g_ref, h_ref, wmu_ref, bmu_ref, wlv_ref, blv_ref,
             mu_ref, lv_ref):
    dis = _dis_of(h_ref)
    agg = s_ref[0] + s_ref[1]
    a3 = dis * agg + dis * g_ref[...]
    mu_ref[...] = jnp.dot(a3, wmu_ref[...], preferred_element_type=jnp.float32) + bmu_ref[...]
    lv_ref[...] = jnp.dot(a3, wlv_ref[...], preferred_element_type=jnp.float32) + blv_ref[...]


def _row_spec(width):
    return pl.BlockSpec((ROWB, width), lambda i: (i, 0))


def _part_spec(width):
    return pl.BlockSpec((2, ROWB, width), lambda i: (0, i, 0))


def _full_spec(r, cmax):
    return pl.BlockSpec((r, cmax), lambda i: (0, 0))


@jax.jit
def kernel(x, edge_index, W1, b1, W2, b2, W_mu, b_mu, W_logvar, b_logvar):
    src = edge_index[0]
    dst = edge_index[1]
    pad = (N_NODES
           + jnp.arange(EP - N_EDGES, dtype=jnp.int32) % (NP - N_NODES))
    src3 = jnp.concatenate([src, pad]).reshape(NW, CPT, CHUNK)
    dst3 = jnp.concatenate([dst, pad]).reshape(NW, CPT, CHUNK)
    b1r = b1.reshape(1, -1)
    b2r = b2.reshape(1, -1)
    bmur = b_mu.reshape(1, -1)
    blvr = b_logvar.reshape(1, -1)

    hist = _hist(dst3)

    g1 = pl.pallas_call(
        _k1_body,
        grid=(GRID,),
        in_specs=[_row_spec(D_IN), _full_spec(D_IN, D_HID), _part_spec(16)],
        out_specs=_row_spec(D_HID),
        out_shape=jax.ShapeDtypeStruct((NP, D_HID), jnp.float32),
    )(x, W1, hist)

    s1 = _agg(g1, src3, dst3)

    g2 = pl.pallas_call(
        functools.partial(_k23_body, apply_w=True),
        grid=(GRID,),
        in_specs=[_part_spec(D_HID), _row_spec(D_HID), _part_spec(16),
                  _full_spec(D_HID, D_HID), _full_spec(1, D_HID)],
        out_specs=_row_spec(D_HID),
        out_shape=jax.ShapeDtypeStruct((NP, D_HID), jnp.float32),
    )(s1, g1, hist, W2, b1r)

    s2 = _agg(g2, src3, dst3)

    g3 = pl.pallas_call(
        functools.partial(_k23_body, apply_w=False),
        grid=(GRID,),
        in_specs=[_part_spec(D_HID), _row_spec(D_HID), _part_spec(16),
                  _full_spec(D_HID, D_HID), _full_spec(1, D_HID)],
        out_specs=_row_spec(D_HID),
        out_shape=jax.ShapeDtypeStruct((NP, D_HID), jnp.float32),
    )(s2, g2, hist, W2, b2r)

    s3 = _agg(g3, src3, dst3)

    mu, lv = pl.pallas_call(
        _k4_body,
        grid=(GRID,),
        in_specs=[_part_spec(D_HID), _row_spec(D_HID), _part_spec(16),
                  _full_spec(D_HID, D_OUT), _full_spec(1, D_OUT),
                  _full_spec(D_HID, D_OUT), _full_spec(1, D_OUT)],
        out_specs=[_row_spec(D_OUT), _row_spec(D_OUT)],
        out_shape=[jax.ShapeDtypeStruct((N_NODES, D_OUT), jnp.float32)] * 2,
    )(s3, g3, hist, W_mu, bmur, W_logvar, blvr)

    return (mu, lv)


# speedup vs baseline: 1.0974x; 1.0047x over previous
import functools
import jax
import jax.numpy as jnp
from jax import lax
from jax.experimental import pallas as pl
from jax.experimental.pallas import tpu as pltpu
from jax.experimental.pallas import tpu_sc as plsc

N_NODES = 10000
N_EDGES = 320000
D_IN = 128
D_HID = 128
D_OUT = 64

NP = 10240
NW = 32
CHUNK = 128
CPT = 80
EPT = CPT * CHUNK
EP = NW * EPT
HCPT = CPT // 2
RPT = NP // 16
ROWB = 5000
GRID = N_NODES // ROWB


def _mesh():
    return plsc.VectorSubcoreMesh(core_axis_name="c", subcore_axis_name="s")


def _zero_vmem(buf, nrows, ncols):
    zeros = jnp.zeros((16,), jnp.float32)

    def zrow(i, _):
        def zcol(j, _):
            buf[i, pl.ds(j * 16, 16)] = zeros
            return 0
        return lax.fori_loop(0, ncols // 16, zcol, 0)

    lax.fori_loop(0, nrows, zrow, 0)


def _hist_body(dst_hbm, out_hbm, acc, dst_v, buf):
    c = lax.axis_index("c")
    s = lax.axis_index("s")
    wid = c * 16 + s
    base = s * RPT
    _zero_vmem(buf, CHUNK, 16)

    def zacc(k, _):
        pltpu.sync_copy(buf, acc.at[pl.ds(base + k * CHUNK, CHUNK)])
        return 0
    lax.fori_loop(0, RPT // CHUNK, zacc, 0)

    ones = jnp.ones((16,), jnp.float32)

    def orow(i, _):
        buf[i, pl.ds(0, 16)] = ones
        return 0
    lax.fori_loop(0, CHUNK, orow, 0)

    pltpu.sync_copy(dst_hbm.at[wid], dst_v)
    plsc.subcore_barrier()

    def step(j, _):
        pltpu.sync_copy(buf, acc.at[dst_v.at[j]], add=True)
        return 0
    lax.fori_loop(0, CPT, step, 0)
    plsc.subcore_barrier()

    def wout(k, _):
        pltpu.sync_copy(acc.at[pl.ds(base + k * CHUNK, CHUNK)],
                        out_hbm.at[c, pl.ds(base + k * CHUNK, CHUNK)])
        return 0
    lax.fori_loop(0, RPT // CHUNK, wout, 0)


@functools.partial(
    pl.kernel,
    out_type=jax.ShapeDtypeStruct((2, NP, 16), jnp.float32),
    mesh=_mesh(),
    scratch_types=[
        pltpu.VMEM_SHARED((NP, 16), jnp.float32),
        pltpu.VMEM((CPT, CHUNK), jnp.int32),
        pltpu.VMEM((CHUNK, 16), jnp.float32),
    ],
)
def _hist(dst_hbm, out_hbm, acc, dst_v, buf):
    _hist_body(dst_hbm, out_hbm, acc, dst_v, buf)


def _agg_body(g_hbm, src_hbm, dst_hbm, out_hbm, acc,
              src_v, dst_v, rows0, rows1, gs0, gs1, ss0, ss1):
    c = lax.axis_index("c")
    s = lax.axis_index("s")
    wid = c * 16 + s
    base = s * RPT
    _zero_vmem(rows0, CHUNK, D_HID)

    def zacc(k, _):
        pltpu.sync_copy(rows0, acc.at[pl.ds(base + k * CHUNK, CHUNK)])
        return 0
    lax.fori_loop(0, RPT // CHUNK, zacc, 0)

    plsc.subcore_barrier()

    def half(h, _):
        pltpu.sync_copy(src_hbm.at[wid, pl.ds(h * HCPT, HCPT)], src_v)
        pltpu.sync_copy(dst_hbm.at[wid, pl.ds(h * HCPT, HCPT)], dst_v)
        pltpu.async_copy(g_hbm.at[src_v.at[0]], rows0, gs0)
        pltpu.async_copy(g_hbm.at[src_v.at[1]], rows1, gs1)

        def step(i, _):
            j0 = 2 * i
            pltpu.make_async_copy(g_hbm.at[src_v.at[j0]], rows0, gs0).wait()
            pltpu.sync_copy(rows0, acc.at[dst_v.at[j0]], add=True)

            @pl.when(i < HCPT // 2 - 1)
            def _():
                pltpu.async_copy(g_hbm.at[src_v.at[j0 + 2]], rows0, gs0)

            pltpu.make_async_copy(g_hbm.at[src_v.at[j0 + 1]], rows1, gs1).wait()
            pltpu.sync_copy(rows1, acc.at[dst_v.at[j0 + 1]], add=True)

            @pl.when(i < HCPT // 2 - 1)
            def _():
                pltpu.async_copy(g_hbm.at[src_v.at[j0 + 3]], rows1, gs1)
            return 0
        lax.fori_loop(0, HCPT // 2, step, 0)
        return 0
    lax.fori_loop(0, 2, half, 0)
    plsc.subcore_barrier()

    def wout(k, _):
        pltpu.sync_copy(acc.at[pl.ds(base + k * CHUNK, CHUNK)],
                        out_hbm.at[c, pl.ds(base + k * CHUNK, CHUNK)])
        return 0
    lax.fori_loop(0, RPT // CHUNK, wout, 0)


@functools.partial(
    pl.kernel,
    out_type=jax.ShapeDtypeStruct((2, NP, D_HID), jnp.float32),
    mesh=_mesh(),
    scratch_types=[
        pltpu.VMEM_SHARED((NP, D_HID), jnp.float32),
        pltpu.VMEM((HCPT, CHUNK), jnp.int32),
        pltpu.VMEM((HCPT, CHUNK), jnp.int32),
        pltpu.VMEM((CHUNK, D_HID), jnp.float32),
        pltpu.VMEM((CHUNK, D_HID), jnp.float32),
        pltpu.SemaphoreType.DMA,
        pltpu.SemaphoreType.DMA,
        pltpu.SemaphoreType.DMA,
        pltpu.SemaphoreType.DMA,
    ],
)
def _agg(g_hbm, src_hbm, dst_hbm, out_hbm, acc,
         src_v, dst_v, rows0, rows1, gs0, gs1, ss0, ss1):
    _agg_body(g_hbm, src_hbm, dst_hbm, out_hbm, acc,
              src_v, dst_v, rows0, rows1, gs0, gs1, ss0, ss1)


def _dis_of(h_ref):
    deg = h_ref[0, :, 0:1] + h_ref[1, :, 0:1] + 1.0
    return lax.rsqrt(deg)


def _k1_body(x_ref, w_ref, h_ref, g_ref):
    dis = _dis_of(h_ref)
    p = jnp.dot(x_ref[...], w_ref[...], preferred_element_type=jnp.float32)
    g_ref[...] = dis * p


def _k23_body(s_ref, g_ref, h_ref, w_ref, b_ref, gout_ref, apply_w):
    dis = _dis_of(h_ref)
    agg = s_ref[0] + s_ref[1]
    h = jnp.maximum(dis * agg + dis * g_ref[...] + b_ref[...], 0.0)
    if apply_w:
        pnew = jnp.dot(h, w_ref[...], preferred_element_type=jnp.float32)
    else:
        pnew = h
    gout_ref[...] = dis * pnew


def _k4_body(s_ref, g_ref, h_ref, wmu_ref, bmu_ref, wlv_ref, blv_ref,
             mu_ref, lv_ref):
    dis = _dis_of(h_ref)
    agg = s_ref[0] + s_ref[1]
    a3 = dis * agg + dis * g_ref[...]
    mu_ref[...] = jnp.dot(a3, wmu_ref[...], preferred_element_type=jnp.float32) + bmu_ref[...]
    lv_ref[...] = jnp.dot(a3, wlv_ref[...], preferred_element_type=jnp.float32) + blv_ref[...]


def _row_spec(width):
    return pl.BlockSpec((ROWB, width), lambda i: (i, 0))


def _part_spec(width):
    return pl.BlockSpec((2, ROWB, width), lambda i: (0, i, 0))


def _full_spec(r, cmax):
    return pl.BlockSpec((r, cmax), lambda i: (0, 0))


@jax.jit
def kernel(x, edge_index, W1, b1, W2, b2, W_mu, b_mu, W_logvar, b_logvar):
    src = edge_index[0]
    dst = edge_index[1]
    pad = (N_NODES
           + jnp.arange(EP - N_EDGES, dtype=jnp.int32) % (NP - N_NODES))
    src3 = jnp.concatenate([src, pad]).reshape(NW, CPT, CHUNK)
    dst3 = jnp.concatenate([dst, pad]).reshape(NW, CPT, CHUNK)
    b1r = b1.reshape(1, -1)
    b2r = b2.reshape(1, -1)
    bmur = b_mu.reshape(1, -1)
    blvr = b_logvar.reshape(1, -1)

    hist = _hist(dst3)

    g1 = pl.pallas_call(
        _k1_body,
        grid=(GRID,),
        in_specs=[_row_spec(D_IN), _full_spec(D_IN, D_HID), _part_spec(16)],
        out_specs=_row_spec(D_HID),
        out_shape=jax.ShapeDtypeStruct((NP, D_HID), jnp.float32),
    )(x, W1, hist)

    s1 = _agg(g1, src3, dst3)

    g2 = pl.pallas_call(
        functools.partial(_k23_body, apply_w=True),
        grid=(GRID,),
        in_specs=[_part_spec(D_HID), _row_spec(D_HID), _part_spec(16),
                  _full_spec(D_HID, D_HID), _full_spec(1, D_HID)],
        out_specs=_row_spec(D_HID),
        out_shape=jax.ShapeDtypeStruct((NP, D_HID), jnp.float32),
    )(s1, g1, hist, W2, b1r)

    s2 = _agg(g2, src3, dst3)

    g3 = pl.pallas_call(
        functools.partial(_k23_body, apply_w=False),
        grid=(GRID,),
        in_specs=[_part_spec(D_HID), _row_spec(D_HID), _part_spec(16),
                  _full_spec(D_HID, D_HID), _full_spec(1, D_HID)],
        out_specs=_row_spec(D_HID),
        out_shape=jax.ShapeDtypeStruct((NP, D_HID), jnp.float32),
    )(s2, g2, hist, W2, b2r)

    s3 = _agg(g3, src3, dst3)

    mu, lv = pl.pallas_call(
        _k4_body,
        grid=(GRID,),
        in_specs=[_part_spec(D_HID), _row_spec(D_HID), _part_spec(16),
                  _full_spec(D_HID, D_OUT), _full_spec(1, D_OUT),
                  _full_spec(D_HID, D_OUT), _full_spec(1, D_OUT)],
        out_specs=[_row_spec(D_OUT), _row_spec(D_OUT)],
        out_shape=[jax.ShapeDtypeStruct((N_NODES, D_OUT), jnp.float32)] * 2,
    )(s3, g3, hist, W_mu, bmur, W_logvar, blvr)

    return (mu, lv)


# speedup vs baseline: 1.1033x; 1.0053x over previous
import functools
import jax
import jax.numpy as jnp
from jax import lax
from jax.experimental import pallas as pl
from jax.experimental.pallas import tpu as pltpu
from jax.experimental.pallas import tpu_sc as plsc

N_NODES = 10000
N_EDGES = 320000
D_IN = 128
D_HID = 128
D_OUT = 64

NP = 10240
NW = 32
CHUNK = 128
CPT = 80
EPT = CPT * CHUNK
EP = NW * EPT
HCPT = CPT // 2
RPT = NP // 16
ROWB = 5000
GRID = N_NODES // ROWB


def _mesh():
    return plsc.VectorSubcoreMesh(core_axis_name="c", subcore_axis_name="s")


def _zero_vmem(buf, nrows, ncols):
    zeros = jnp.zeros((16,), jnp.float32)

    def zrow(i, _):
        def zcol(j, _):
            buf[i, pl.ds(j * 16, 16)] = zeros
            return 0
        return lax.fori_loop(0, ncols // 16, zcol, 0)

    lax.fori_loop(0, nrows, zrow, 0)


def _hist_body(dst_hbm, out_hbm, acc, dst_v, buf, hsem):
    c = lax.axis_index("c")
    s = lax.axis_index("s")
    wid = c * 16 + s
    base = s * RPT
    _zero_vmem(buf, CHUNK, 16)

    def zacc(k, _):
        pltpu.sync_copy(buf, acc.at[pl.ds(base + k * CHUNK, CHUNK)])
        return 0
    lax.fori_loop(0, RPT // CHUNK, zacc, 0)

    ones = jnp.ones((16,), jnp.float32)

    def orow(i, _):
        buf[i, pl.ds(0, 16)] = ones
        return 0
    lax.fori_loop(0, CHUNK, orow, 0)

    pltpu.sync_copy(dst_hbm.at[wid], dst_v)
    plsc.subcore_barrier()

    def step(i, _):
        def fire(j, _):
            pltpu.async_copy(buf, acc.at[dst_v.at[i * 8 + j]], hsem, add=True)
            return 0
        lax.fori_loop(0, 8, fire, 0)

        def drain(j, _):
            pltpu.make_async_copy(buf, acc.at[dst_v.at[i * 8 + j]], hsem).wait()
            return 0
        lax.fori_loop(0, 8, drain, 0)
        return 0
    lax.fori_loop(0, CPT // 8, step, 0)
    plsc.subcore_barrier()

    def wout(k, _):
        pltpu.sync_copy(acc.at[pl.ds(base + k * CHUNK, CHUNK)],
                        out_hbm.at[c, pl.ds(base + k * CHUNK, CHUNK)])
        return 0
    lax.fori_loop(0, RPT // CHUNK, wout, 0)


@functools.partial(
    pl.kernel,
    out_type=jax.ShapeDtypeStruct((2, NP, 16), jnp.float32),
    mesh=_mesh(),
    scratch_types=[
        pltpu.VMEM_SHARED((NP, 16), jnp.float32),
        pltpu.VMEM((CPT, CHUNK), jnp.int32),
        pltpu.VMEM((CHUNK, 16), jnp.float32),
        pltpu.SemaphoreType.DMA,
    ],
)
def _hist(dst_hbm, out_hbm, acc, dst_v, buf, hsem):
    _hist_body(dst_hbm, out_hbm, acc, dst_v, buf, hsem)


def _agg_body(g_hbm, src_hbm, dst_hbm, out_hbm, acc,
              src_v, dst_v, rows0, rows1, gs0, gs1, ss0, ss1):
    c = lax.axis_index("c")
    s = lax.axis_index("s")
    wid = c * 16 + s
    base = s * RPT
    _zero_vmem(rows0, CHUNK, D_HID)

    def zacc(k, _):
        pltpu.sync_copy(rows0, acc.at[pl.ds(base + k * CHUNK, CHUNK)])
        return 0
    lax.fori_loop(0, RPT // CHUNK, zacc, 0)

    plsc.subcore_barrier()

    def half(h, _):
        pltpu.sync_copy(src_hbm.at[wid, pl.ds(h * HCPT, HCPT)], src_v)
        pltpu.sync_copy(dst_hbm.at[wid, pl.ds(h * HCPT, HCPT)], dst_v)
        pltpu.async_copy(g_hbm.at[src_v.at[0]], rows0, gs0)
        pltpu.async_copy(g_hbm.at[src_v.at[1]], rows1, gs1)

        def step(i, _):
            j0 = 2 * i
            pltpu.make_async_copy(g_hbm.at[src_v.at[j0]], rows0, gs0).wait()
            pltpu.sync_copy(rows0, acc.at[dst_v.at[j0]], add=True)

            @pl.when(i < HCPT // 2 - 1)
            def _():
                pltpu.async_copy(g_hbm.at[src_v.at[j0 + 2]], rows0, gs0)

            pltpu.make_async_copy(g_hbm.at[src_v.at[j0 + 1]], rows1, gs1).wait()
            pltpu.sync_copy(rows1, acc.at[dst_v.at[j0 + 1]], add=True)

            @pl.when(i < HCPT // 2 - 1)
            def _():
                pltpu.async_copy(g_hbm.at[src_v.at[j0 + 3]], rows1, gs1)
            return 0
        lax.fori_loop(0, HCPT // 2, step, 0)
        return 0
    lax.fori_loop(0, 2, half, 0)
    plsc.subcore_barrier()

    def wout(k, _):
        pltpu.sync_copy(acc.at[pl.ds(base + k * CHUNK, CHUNK)],
                        out_hbm.at[c, pl.ds(base + k * CHUNK, CHUNK)])
        return 0
    lax.fori_loop(0, RPT // CHUNK, wout, 0)


@functools.partial(
    pl.kernel,
    out_type=jax.ShapeDtypeStruct((2, NP, D_HID), jnp.float32),
    mesh=_mesh(),
    scratch_types=[
        pltpu.VMEM_SHARED((NP, D_HID), jnp.float32),
        pltpu.VMEM((HCPT, CHUNK), jnp.int32),
        pltpu.VMEM((HCPT, CHUNK), jnp.int32),
        pltpu.VMEM((CHUNK, D_HID), jnp.float32),
        pltpu.VMEM((CHUNK, D_HID), jnp.float32),
        pltpu.SemaphoreType.DMA,
        pltpu.SemaphoreType.DMA,
        pltpu.SemaphoreType.DMA,
        pltpu.SemaphoreType.DMA,
    ],
)
def _agg(g_hbm, src_hbm, dst_hbm, out_hbm, acc,
         src_v, dst_v, rows0, rows1, gs0, gs1, ss0, ss1):
    _agg_body(g_hbm, src_hbm, dst_hbm, out_hbm, acc,
              src_v, dst_v, rows0, rows1, gs0, gs1, ss0, ss1)


def _dis_of(h_ref):
    deg = h_ref[0, :, 0:1] + h_ref[1, :, 0:1] + 1.0
    return lax.rsqrt(deg)


def _k1a_body(x_ref, w_ref, q_ref):
    q_ref[...] = jnp.dot(x_ref[...], w_ref[...],
                         preferred_element_type=jnp.float32)


def _k1b_body(q_ref, h_ref, g_ref):
    g_ref[...] = _dis_of(h_ref) * q_ref[...]


def _k23_body(s_ref, g_ref, h_ref, w_ref, b_ref, gout_ref, apply_w):
    dis = _dis_of(h_ref)
    agg = s_ref[0] + s_ref[1]
    h = jnp.maximum(dis * agg + dis * g_ref[...] + b_ref[...], 0.0)
    if apply_w:
        pnew = jnp.dot(h, w_ref[...], preferred_element_type=jnp.float32)
    else:
        pnew = h
    gout_ref[...] = dis * pnew


def _k4_body(s_ref, g_ref, h_ref, wmu_ref, bmu_ref, wlv_ref, blv_ref,
             mu_ref, lv_ref):
    dis = _dis_of(h_ref)
    agg = s_ref[0] + s_ref[1]
    a3 = dis * agg + dis * g_ref[...]
    mu_ref[...] = jnp.dot(a3, wmu_ref[...], preferred_element_type=jnp.float32) + bmu_ref[...]
    lv_ref[...] = jnp.dot(a3, wlv_ref[...], preferred_element_type=jnp.float32) + blv_ref[...]


def _row_spec(width):
    return pl.BlockSpec((ROWB, width), lambda i: (i, 0))


def _part_spec(width):
    return pl.BlockSpec((2, ROWB, width), lambda i: (0, i, 0))


def _full_spec(r, cmax):
    return pl.BlockSpec((r, cmax), lambda i: (0, 0))


@jax.jit
def kernel(x, edge_index, W1, b1, W2, b2, W_mu, b_mu, W_logvar, b_logvar):
    src = edge_index[0]
    dst = edge_index[1]
    pad = (N_NODES
           + jnp.arange(EP - N_EDGES, dtype=jnp.int32) % (NP - N_NODES))
    src3 = jnp.concatenate([src, pad]).reshape(NW, CPT, CHUNK)
    dst3 = jnp.concatenate([dst, pad]).reshape(NW, CPT, CHUNK)
    b1r = b1.reshape(1, -1)
    b2r = b2.reshape(1, -1)
    bmur = b_mu.reshape(1, -1)
    blvr = b_logvar.reshape(1, -1)

    hist = _hist(dst3)

    q1 = pl.pallas_call(
        _k1a_body,
        grid=(GRID,),
        in_specs=[_row_spec(D_IN), _full_spec(D_IN, D_HID)],
        out_specs=_row_spec(D_HID),
        out_shape=jax.ShapeDtypeStruct((NP, D_HID), jnp.float32),
    )(x, W1)

    g1 = pl.pallas_call(
        _k1b_body,
        grid=(GRID,),
        in_specs=[_row_spec(D_HID), _part_spec(16)],
        out_specs=_row_spec(D_HID),
        out_shape=jax.ShapeDtypeStruct((NP, D_HID), jnp.float32),
    )(q1, hist)

    s1 = _agg(g1, src3, dst3)

    g2 = pl.pallas_call(
        functools.partial(_k23_body, apply_w=True),
        grid=(GRID,),
        in_specs=[_part_spec(D_HID), _row_spec(D_HID), _part_spec(16),
                  _full_spec(D_HID, D_HID), _full_spec(1, D_HID)],
        out_specs=_row_spec(D_HID),
        out_shape=jax.ShapeDtypeStruct((NP, D_HID), jnp.float32),
    )(s1, g1, hist, W2, b1r)

    s2 = _agg(g2, src3, dst3)

    g3 = pl.pallas_call(
        functools.partial(_k23_body, apply_w=False),
        grid=(GRID,),
        in_specs=[_part_spec(D_HID), _row_spec(D_HID), _part_spec(16),
                  _full_spec(D_HID, D_HID), _full_spec(1, D_HID)],
        out_specs=_row_spec(D_HID),
        out_shape=jax.ShapeDtypeStruct((NP, D_HID), jnp.float32),
    )(s2, g2, hist, W2, b2r)

    s3 = _agg(g3, src3, dst3)

    mu, lv = pl.pallas_call(
        _k4_body,
        grid=(GRID,),
        in_specs=[_part_spec(D_HID), _row_spec(D_HID), _part_spec(16),
                  _full_spec(D_HID, D_OUT), _full_spec(1, D_OUT),
                  _full_spec(D_HID, D_OUT), _full_spec(1, D_OUT)],
        out_specs=[_row_spec(D_OUT), _row_spec(D_OUT)],
        out_shape=[jax.ShapeDtypeStruct((N_NODES, D_OUT), jnp.float32)] * 2,
    )(s3, g3, hist, W_mu, bmur, W_logvar, blvr)

    return (mu, lv)
